# baseline (speedup 1.0000x reference)
import functools

import jax
import jax.numpy as jnp
from jax import lax
from jax.experimental import pallas as pl
from jax.experimental.pallas import tpu as pltpu
from jax.experimental.pallas import tpu_sc as plsc

N_NODES = 10000
NPAD = 10240
IN_DIM = 128
HID_DIM = 128
NUM_CLASSES = 40
N_EDGES = 320000

NC = 2
NS = 16
NW = NC * NS
EW = N_EDGES // NW
CHUNK = 100
KCH = EW // CHUNK
ROWS_PER_SUB = NPAD // NS
ZR = 40

_MESH = plsc.VectorSubcoreMesh(core_axis_name="c", subcore_axis_name="s")
_SC_PARAMS = pltpu.CompilerParams(use_tc_tiling_on_sc=False,
                                  skip_device_barrier=True)


@functools.partial(
    pl.kernel,
    out_type=jax.ShapeDtypeStruct((NC, NPAD, 8), jnp.float32),
    mesh=_MESH,
    scratch_types=[
        pltpu.VMEM((KCH, CHUNK), jnp.int32),
        pltpu.VMEM((CHUNK, 8), jnp.float32),
        pltpu.VMEM_SHARED((NPAD, 8), jnp.float32),
        pltpu.SemaphoreType.DMA,
        pltpu.SemaphoreType.DMA,
    ],
    compiler_params=_SC_PARAMS,
)
def _sc_degree(dst_hbm, ones_hbm, out_hbm, idx_v, ones_v, acc, sem0, sem1):
    c = lax.axis_index("c")
    s = lax.axis_index("s")
    wid = s * NC + c
    base = s * ROWS_PER_SUB
    pltpu.async_copy(dst_hbm.at[wid], idx_v, sem0)
    pltpu.sync_copy(ones_hbm, ones_v)
    @pl.loop(0, ROWS_PER_SUB, step=ZR)
    def _(r):
        pltpu.sync_copy(ones_v.at[pl.ds(0, ZR)], acc.at[pl.ds(base + r, ZR)])

    pltpu.make_async_copy(dst_hbm.at[wid], idx_v, sem0).wait()
    plsc.subcore_barrier()

    sems = (sem0, sem1)

    @pl.loop(0, KCH, step=2)
    def _(j):
        for b in range(2):
            jj = j + b

            @pl.when(jj >= 2)
            def _():
                pltpu.make_async_copy(
                    ones_v, acc.at[idx_v.at[jj - 2]], sems[b]).wait()

            pltpu.async_copy(ones_v, acc.at[idx_v.at[jj]], sems[b], add=True)

    for b in range(2):
        pltpu.make_async_copy(
            ones_v, acc.at[idx_v.at[KCH - 2 + b]], sems[b]).wait()

    plsc.subcore_barrier()
    pltpu.sync_copy(acc.at[pl.ds(base, ROWS_PER_SUB)],
                    out_hbm.at[c, pl.ds(base, ROWS_PER_SUB)])


def _make_sc_aggregate(dim, chunk, nbuf):
    kch = EW // chunk
    assert kch % nbuf == 0

    @functools.partial(
        pl.kernel,
        out_type=jax.ShapeDtypeStruct((NC, NPAD, dim), jnp.float32),
        mesh=_MESH,
        scratch_types=(
            [pltpu.VMEM((kch, chunk), jnp.int32)] * 2
            + [pltpu.VMEM((chunk, dim), jnp.float32)] * nbuf
            + [pltpu.VMEM_SHARED((NPAD, dim), jnp.float32)]
            + [pltpu.SemaphoreType.DMA] * (2 * nbuf + 1)
        ),
        compiler_params=_SC_PARAMS,
    )
    def agg(h_hbm, src_hbm, dst_hbm, zeros_hbm, out_hbm, *refs):
        src_v, dst_v = refs[0], refs[1]
        gbufs = refs[2:2 + nbuf]
        acc = refs[2 + nbuf]
        gsems = refs[3 + nbuf:3 + 2 * nbuf]
        ssems = refs[3 + 2 * nbuf:3 + 3 * nbuf]
        isem = refs[3 + 3 * nbuf]
        c = lax.axis_index("c")
        s = lax.axis_index("s")
        wid = s * NC + c
        base = s * ROWS_PER_SUB
        pltpu.async_copy(src_hbm.at[wid], src_v, isem)
        pltpu.sync_copy(zeros_hbm, gbufs[0].at[pl.ds(0, ZR)])

        @pl.loop(0, ROWS_PER_SUB, step=ZR)
        def _(r):
            pltpu.sync_copy(gbufs[0].at[pl.ds(0, ZR)],
                            acc.at[pl.ds(base + r, ZR)])

        pltpu.make_async_copy(src_hbm.at[wid], src_v, isem).wait()
        pltpu.sync_copy(dst_hbm.at[wid], dst_v)
        plsc.subcore_barrier()

        if nbuf == 2:
            pltpu.async_copy(h_hbm.at[src_v.at[0]], gbufs[0], gsems[0])

            @pl.loop(0, kch, step=2)
            def _(j):
                for b in range(2):
                    jj = j + b
                    nb = 1 - b

                    @pl.when(jj + 1 < kch)
                    def _():
                        pltpu.async_copy(
                            h_hbm.at[src_v.at[jj + 1]], gbufs[nb], gsems[nb])

                    pltpu.make_async_copy(
                        h_hbm.at[src_v.at[jj]], gbufs[b], gsems[b]).wait()
                    pltpu.sync_copy(gbufs[b], acc.at[dst_v.at[jj]], add=True)
        else:
            pltpu.async_copy(h_hbm.at[src_v.at[0]], gbufs[0], gsems[0])
            pltpu.async_copy(h_hbm.at[src_v.at[1]], gbufs[1], gsems[1])

            @pl.loop(0, kch, step=4)
            def _(j):
                for b in range(4):
                    jj = j + b
                    nb = (b + 2) % 4
                    pltpu.make_async_copy(
                        h_hbm.at[src_v.at[jj]], gbufs[b], gsems[b]).wait()
                    pltpu.async_copy(
                        gbufs[b], acc.at[dst_v.at[jj]], ssems[b], add=True)

                    @pl.when(jj + 2 < kch)
                    def _():
                        @pl.when(jj >= 2)
                        def _():
                            pltpu.make_async_copy(
                                gbufs[nb], acc.at[dst_v.at[jj]],
                                ssems[nb]).wait()

                        pltpu.async_copy(
                            h_hbm.at[src_v.at[jj + 2]], gbufs[nb], gsems[nb])

            for b in range(4):
                pltpu.make_async_copy(
                    gbufs[b], acc.at[dst_v.at[kch - 4 + b]], ssems[b]).wait()

        plsc.subcore_barrier()
        pltpu.sync_copy(acc.at[pl.ds(base, ROWS_PER_SUB)],
                        out_hbm.at[c, pl.ds(base, ROWS_PER_SUB)])

    return agg


_sc_agg128 = _make_sc_aggregate(HID_DIM, 100, 2)
_sc_agg40 = _make_sc_aggregate(NUM_CLASSES, 100, 4)


BR = 2560
GRID = NPAD // BR


def _tc_scale1_body(x_ref, w_ref, deg_ref, h1p_ref, dis_ref):
    d = deg_ref[0][:, 0:1] + deg_ref[1][:, 0:1] - 1.0
    dis = lax.rsqrt(d)
    h = jnp.dot(x_ref[...], w_ref[...], preferred_element_type=jnp.float32)
    h1p_ref[...] = h * dis
    dis_ref[...] = dis


def _tc_scale1(x_pad, w1, degp):
    return pl.pallas_call(
        _tc_scale1_body,
        grid=(GRID,),
        in_specs=[
            pl.BlockSpec((BR, IN_DIM), lambda i: (i, 0)),
            pl.BlockSpec((IN_DIM, HID_DIM), lambda i: (0, 0)),
            pl.BlockSpec((NC, BR, 8), lambda i: (0, i, 0)),
        ],
        out_specs=[
            pl.BlockSpec((BR, HID_DIM), lambda i: (i, 0)),
            pl.BlockSpec((BR, 1), lambda i: (i, 0)),
        ],
        out_shape=[
            jax.ShapeDtypeStruct((NPAD, HID_DIM), jnp.float32),
            jax.ShapeDtypeStruct((NPAD, 1), jnp.float32),
        ],
    )(x_pad, w1, degp)


def _tc_mid_body(p_ref, h1p_ref, dis_ref, b1_ref, w2_ref, h2p_ref):
    agg = p_ref[0] + p_ref[1] + h1p_ref[...]
    out1 = jnp.maximum(agg * dis_ref[...] + b1_ref[...], 0.0)
    h2 = jnp.dot(out1, w2_ref[...], preferred_element_type=jnp.float32)
    h2p_ref[...] = h2 * dis_ref[...]


def _tc_mid(p1, h1p, dis, b1, w2):
    return pl.pallas_call(
        _tc_mid_body,
        grid=(GRID,),
        in_specs=[
            pl.BlockSpec((NC, BR, HID_DIM), lambda i: (0, i, 0)),
            pl.BlockSpec((BR, HID_DIM), lambda i: (i, 0)),
            pl.BlockSpec((BR, 1), lambda i: (i, 0)),
            pl.BlockSpec((1, HID_DIM), lambda i: (0, 0)),
            pl.BlockSpec((HID_DIM, NUM_CLASSES), lambda i: (0, 0)),
        ],
        out_specs=pl.BlockSpec((BR, NUM_CLASSES), lambda i: (i, 0)),
        out_shape=jax.ShapeDtypeStruct((NPAD, NUM_CLASSES), jnp.float32),
    )(p1, h1p, dis, b1, w2)


def _tc_final_body(p_ref, h2p_ref, dis_ref, b2_ref, out_ref):
    agg = p_ref[0] + p_ref[1] + h2p_ref[...]
    out_ref[...] = agg * dis_ref[...] + b2_ref[...]


def _tc_final(p2, h2p, dis, b2):
    return pl.pallas_call(
        _tc_final_body,
        grid=(GRID,),
        in_specs=[
            pl.BlockSpec((NC, BR, NUM_CLASSES), lambda i: (0, i, 0)),
            pl.BlockSpec((BR, NUM_CLASSES), lambda i: (i, 0)),
            pl.BlockSpec((BR, 1), lambda i: (i, 0)),
            pl.BlockSpec((1, NUM_CLASSES), lambda i: (0, 0)),
        ],
        out_specs=pl.BlockSpec((BR, NUM_CLASSES), lambda i: (i, 0)),
        out_shape=jax.ShapeDtypeStruct((NPAD, NUM_CLASSES), jnp.float32),
    )(p2, h2p, dis, b2)


def kernel(x, edge_index, W1, b1, W2, b2):
    ei = edge_index.astype(jnp.int32)
    src100 = ei[0].reshape(NW, EW // 100, 100)
    dst100 = ei[1].reshape(NW, EW // 100, 100)

    x_pad = jnp.pad(x, ((0, NPAD - N_NODES), (0, 0)))
    ones8 = jnp.ones((CHUNK, 8), jnp.float32)
    zeros128 = jnp.zeros((ZR, HID_DIM), jnp.float32)
    zeros40 = jnp.zeros((ZR, NUM_CLASSES), jnp.float32)
    b1r = b1.reshape(1, HID_DIM)
    b2r = b2.reshape(1, NUM_CLASSES)

    degp = _sc_degree(dst100, ones8)
    h1p, dis = _tc_scale1(x_pad, W1, degp)
    p1 = _sc_agg128(h1p, src100, dst100, zeros128)
    h2p = _tc_mid(p1, h1p, dis, b1r, W2)
    p2 = _sc_agg40(h2p, src100, dst100, zeros40)
    out = _tc_final(p2, h2p, dis, b2r)
    return out[:N_NODES]


# speedup vs baseline: 1.0007x; 1.0007x over previous
import functools

import jax
import jax.numpy as jnp
from jax import lax
from jax.experimental import pallas as pl
from jax.experimental.pallas import tpu as pltpu
from jax.experimental.pallas import tpu_sc as plsc

N_NODES = 10000
NPAD = 10240
IN_DIM = 128
HID_DIM = 128
NUM_CLASSES = 40
N_EDGES = 320000

NC = 2
NS = 16
NW = NC * NS
EW = N_EDGES // NW
CHUNK = 100
KCH = EW // CHUNK
ROWS_PER_SUB = NPAD // NS
ZR = 40

_MESH = plsc.VectorSubcoreMesh(core_axis_name="c", subcore_axis_name="s")
_SC_PARAMS = pltpu.CompilerParams(use_tc_tiling_on_sc=False)


@functools.partial(
    pl.kernel,
    out_type=jax.ShapeDtypeStruct((NC, NPAD, 8), jnp.float32),
    mesh=_MESH,
    scratch_types=[
        pltpu.VMEM((KCH, CHUNK), jnp.int32),
        pltpu.VMEM((CHUNK, 8), jnp.float32),
        pltpu.VMEM_SHARED((NPAD, 8), jnp.float32),
        pltpu.SemaphoreType.DMA,
        pltpu.SemaphoreType.DMA,
    ],
    compiler_params=_SC_PARAMS,
)
def _sc_degree(dst_hbm, ones_hbm, out_hbm, idx_v, ones_v, acc, sem0, sem1):
    c = lax.axis_index("c")
    s = lax.axis_index("s")
    wid = s * NC + c
    base = s * ROWS_PER_SUB
    pltpu.async_copy(dst_hbm.at[wid], idx_v, sem0)
    pltpu.sync_copy(ones_hbm, ones_v)
    @pl.loop(0, ROWS_PER_SUB, step=ZR)
    def _(r):
        pltpu.sync_copy(ones_v.at[pl.ds(0, ZR)], acc.at[pl.ds(base + r, ZR)])

    pltpu.make_async_copy(dst_hbm.at[wid], idx_v, sem0).wait()
    plsc.subcore_barrier()

    sems = (sem0, sem1)

    @pl.loop(0, KCH, step=2)
    def _(j):
        for b in range(2):
            jj = j + b

            @pl.when(jj >= 2)
            def _():
                pltpu.make_async_copy(
                    ones_v, acc.at[idx_v.at[jj - 2]], sems[b]).wait()

            pltpu.async_copy(ones_v, acc.at[idx_v.at[jj]], sems[b], add=True)

    for b in range(2):
        pltpu.make_async_copy(
            ones_v, acc.at[idx_v.at[KCH - 2 + b]], sems[b]).wait()

    plsc.subcore_barrier()
    pltpu.sync_copy(acc.at[pl.ds(base, ROWS_PER_SUB)],
                    out_hbm.at[c, pl.ds(base, ROWS_PER_SUB)])


def _make_sc_aggregate(dim, chunk, nbuf):
    kch = EW // chunk
    assert kch % nbuf == 0

    @functools.partial(
        pl.kernel,
        out_type=jax.ShapeDtypeStruct((NC, NPAD, dim), jnp.float32),
        mesh=_MESH,
        scratch_types=(
            [pltpu.VMEM((kch, chunk), jnp.int32)] * 2
            + [pltpu.VMEM((chunk, dim), jnp.float32)] * nbuf
            + [pltpu.VMEM_SHARED((NPAD, dim), jnp.float32)]
            + [pltpu.SemaphoreType.DMA] * (2 * nbuf + 1)
        ),
        compiler_params=_SC_PARAMS,
    )
    def agg(h_hbm, src_hbm, dst_hbm, zeros_hbm, out_hbm, *refs):
        src_v, dst_v = refs[0], refs[1]
        gbufs = refs[2:2 + nbuf]
        acc = refs[2 + nbuf]
        gsems = refs[3 + nbuf:3 + 2 * nbuf]
        ssems = refs[3 + 2 * nbuf:3 + 3 * nbuf]
        isem = refs[3 + 3 * nbuf]
        c = lax.axis_index("c")
        s = lax.axis_index("s")
        wid = s * NC + c
        base = s * ROWS_PER_SUB
        pltpu.async_copy(src_hbm.at[wid], src_v, isem)
        pltpu.sync_copy(zeros_hbm, gbufs[0].at[pl.ds(0, ZR)])

        @pl.loop(0, ROWS_PER_SUB, step=ZR)
        def _(r):
            pltpu.sync_copy(gbufs[0].at[pl.ds(0, ZR)],
                            acc.at[pl.ds(base + r, ZR)])

        pltpu.make_async_copy(src_hbm.at[wid], src_v, isem).wait()
        pltpu.sync_copy(dst_hbm.at[wid], dst_v)
        plsc.subcore_barrier()

        if nbuf == 2:
            pltpu.async_copy(h_hbm.at[src_v.at[0]], gbufs[0], gsems[0])

            @pl.loop(0, kch, step=2)
            def _(j):
                for b in range(2):
                    jj = j + b
                    nb = 1 - b

                    @pl.when(jj + 1 < kch)
                    def _():
                        pltpu.async_copy(
                            h_hbm.at[src_v.at[jj + 1]], gbufs[nb], gsems[nb])

                    pltpu.make_async_copy(
                        h_hbm.at[src_v.at[jj]], gbufs[b], gsems[b]).wait()
                    pltpu.sync_copy(gbufs[b], acc.at[dst_v.at[jj]], add=True)
        else:
            pltpu.async_copy(h_hbm.at[src_v.at[0]], gbufs[0], gsems[0])
            pltpu.async_copy(h_hbm.at[src_v.at[1]], gbufs[1], gsems[1])

            @pl.loop(0, kch, step=4)
            def _(j):
                for b in range(4):
                    jj = j + b
                    nb = (b + 2) % 4
                    pltpu.make_async_copy(
                        h_hbm.at[src_v.at[jj]], gbufs[b], gsems[b]).wait()
                    pltpu.async_copy(
                        gbufs[b], acc.at[dst_v.at[jj]], ssems[b], add=True)

                    @pl.when(jj + 2 < kch)
                    def _():
                        @pl.when(jj >= 2)
                        def _():
                            pltpu.make_async_copy(
                                gbufs[nb], acc.at[dst_v.at[jj]],
                                ssems[nb]).wait()

                        pltpu.async_copy(
                            h_hbm.at[src_v.at[jj + 2]], gbufs[nb], gsems[nb])

            for b in range(4):
                pltpu.make_async_copy(
                    gbufs[b], acc.at[dst_v.at[kch - 4 + b]], ssems[b]).wait()

        plsc.subcore_barrier()
        pltpu.sync_copy(acc.at[pl.ds(base, ROWS_PER_SUB)],
                        out_hbm.at[c, pl.ds(base, ROWS_PER_SUB)])

    return agg


_sc_agg128 = _make_sc_aggregate(HID_DIM, 100, 2)
_sc_agg40 = _make_sc_aggregate(NUM_CLASSES, 100, 4)


BR = 2560
GRID = NPAD // BR


def _tc_scale1_body(x_ref, w_ref, deg_ref, h1p_ref, dis_ref):
    d = deg_ref[0][:, 0:1] + deg_ref[1][:, 0:1] - 1.0
    dis = lax.rsqrt(d)
    h = jnp.dot(x_ref[...], w_ref[...], preferred_element_type=jnp.float32)
    h1p_ref[...] = h * dis
    dis_ref[...] = dis


def _tc_scale1(x_pad, w1, degp):
    return pl.pallas_call(
        _tc_scale1_body,
        grid=(GRID,),
        in_specs=[
            pl.BlockSpec((BR, IN_DIM), lambda i: (i, 0)),
            pl.BlockSpec((IN_DIM, HID_DIM), lambda i: (0, 0)),
            pl.BlockSpec((NC, BR, 8), lambda i: (0, i, 0)),
        ],
        out_specs=[
            pl.BlockSpec((BR, HID_DIM), lambda i: (i, 0)),
            pl.BlockSpec((BR, 1), lambda i: (i, 0)),
        ],
        out_shape=[
            jax.ShapeDtypeStruct((NPAD, HID_DIM), jnp.float32),
            jax.ShapeDtypeStruct((NPAD, 1), jnp.float32),
        ],
    )(x_pad, w1, degp)


def _tc_mid_body(p_ref, h1p_ref, dis_ref, b1_ref, w2_ref, h2p_ref):
    agg = p_ref[0] + p_ref[1] + h1p_ref[...]
    out1 = jnp.maximum(agg * dis_ref[...] + b1_ref[...], 0.0)
    h2 = jnp.dot(out1, w2_ref[...], preferred_element_type=jnp.float32)
    h2p_ref[...] = h2 * dis_ref[...]


def _tc_mid(p1, h1p, dis, b1, w2):
    return pl.pallas_call(
        _tc_mid_body,
        grid=(GRID,),
        in_specs=[
            pl.BlockSpec((NC, BR, HID_DIM), lambda i: (0, i, 0)),
            pl.BlockSpec((BR, HID_DIM), lambda i: (i, 0)),
            pl.BlockSpec((BR, 1), lambda i: (i, 0)),
            pl.BlockSpec((1, HID_DIM), lambda i: (0, 0)),
            pl.BlockSpec((HID_DIM, NUM_CLASSES), lambda i: (0, 0)),
        ],
        out_specs=pl.BlockSpec((BR, NUM_CLASSES), lambda i: (i, 0)),
        out_shape=jax.ShapeDtypeStruct((NPAD, NUM_CLASSES), jnp.float32),
    )(p1, h1p, dis, b1, w2)


def _tc_final_body(p_ref, h2p_ref, dis_ref, b2_ref, out_ref):
    agg = p_ref[0] + p_ref[1] + h2p_ref[...]
    out_ref[...] = agg * dis_ref[...] + b2_ref[...]


def _tc_final(p2, h2p, dis, b2):
    return pl.pallas_call(
        _tc_final_body,
        grid=(GRID,),
        in_specs=[
            pl.BlockSpec((NC, BR, NUM_CLASSES), lambda i: (0, i, 0)),
            pl.BlockSpec((BR, NUM_CLASSES), lambda i: (i, 0)),
            pl.BlockSpec((BR, 1), lambda i: (i, 0)),
            pl.BlockSpec((1, NUM_CLASSES), lambda i: (0, 0)),
        ],
        out_specs=pl.BlockSpec((BR, NUM_CLASSES), lambda i: (i, 0)),
        out_shape=jax.ShapeDtypeStruct((NPAD, NUM_CLASSES), jnp.float32),
    )(p2, h2p, dis, b2)


def kernel(x, edge_index, W1, b1, W2, b2):
    ei = edge_index.astype(jnp.int32)
    src100 = ei[0].reshape(NW, EW // 100, 100)
    dst100 = ei[1].reshape(NW, EW // 100, 100)

    x_pad = jnp.pad(x, ((0, NPAD - N_NODES), (0, 0)))

    ones8 = jnp.ones((CHUNK, 8), jnp.float32)
    zeros128 = jnp.zeros((ZR, HID_DIM), jnp.float32)
    zeros40 = jnp.zeros((ZR, NUM_CLASSES), jnp.float32)
    b1r = b1.reshape(1, HID_DIM)
    b2r = b2.reshape(1, NUM_CLASSES)

    degp = _sc_degree(dst100, ones8)
    h1p, dis = _tc_scale1(x_pad, W1, degp)
    p1 = _sc_agg128(h1p, src100, dst100, zeros128)
    h2p = _tc_mid(p1, h1p, dis, b1r, W2)
    p2 = _sc_agg40(h2p, src100, dst100, zeros40)
    out = _tc_final(p2, h2p, dis, b2r)
    return out[:N_NODES]
